# Initial kernel scaffold; baseline (speedup 1.0000x reference)
import jax
import jax.numpy as jnp
from jax.experimental import pallas as pl


def kernel(img, w0, b0, w1, b1, w2, b2, wl1, bl1, wh, bh):
    raise NotImplementedError("write your pallas kernel here")


# speedup vs baseline: 1.0512x; 1.0512x over previous
import jax
import jax.numpy as jnp
from jax.experimental import pallas as pl
from jax.experimental.pallas import tpu as pltpu

A_DIM = 5
IMG_HW = 30
G = 32
RPI = G * G
OFFS = tuple(ky * G + kx for ky in range(3) for kx in range(3))
SH = 72

CTB = 8
MTB = 256


def _conv_kernel(x_ref, w0_ref, b0_ref, w1_ref, b1_ref, w2_ref, b2_ref,
                 out_ref, y0f_ref, y0b_ref, y1f_ref, y1b_ref):
    R = out_ref.shape[0]
    L0 = y0f_ref.shape[0]
    L1 = y1f_ref.shape[0]
    L2 = L1 - SH

    y0f_ref[...] = x_ref[pl.ds(OFFS[0], L0), :] * w0_ref[pl.ds(0, 1), :]
    for t in range(1, 9):
        y0f_ref[...] += x_ref[pl.ds(OFFS[t], L0), :] * w0_ref[pl.ds(t, 1), :]
    y0b_ref[...] = jnp.maximum(y0f_ref[...] + b0_ref[...],
                               0.0).astype(jnp.bfloat16)

    y1f_ref[...] = jnp.dot(y0b_ref[pl.ds(OFFS[0], L1), :], w1_ref[0],
                           preferred_element_type=jnp.float32)
    for t in range(1, 9):
        y1f_ref[...] += jnp.dot(y0b_ref[pl.ds(OFFS[t], L1), :], w1_ref[t],
                                preferred_element_type=jnp.float32)
    y1b_ref[...] = jnp.maximum(y1f_ref[...] + b1_ref[...],
                               0.0).astype(jnp.bfloat16)

    acc = jnp.dot(y1b_ref[pl.ds(OFFS[0], L2), :], w2_ref[0],
                  preferred_element_type=jnp.float32)
    for t in range(1, 9):
        acc += jnp.dot(y1b_ref[pl.ds(OFFS[t], L2), :], w2_ref[t],
                       preferred_element_type=jnp.float32)
    out_ref[pl.ds(0, L2), :] = jnp.maximum(acc + b2_ref[...],
                                           0.0).astype(jnp.bfloat16)
    out_ref[pl.ds(L2, R - L2), :] = jnp.zeros((R - L2, out_ref.shape[1]),
                                              jnp.bfloat16)


def _mlp_kernel(x_ref, wl1_ref, bl1_ref, wh_ref, bh_ref, out_ref):
    h = jnp.dot(x_ref[...], wl1_ref[...], preferred_element_type=jnp.float32)
    h = jnp.maximum(h + bl1_ref[...], 0.0).astype(jnp.bfloat16)
    out_ref[...] = jnp.dot(h, wh_ref[...],
                           preferred_element_type=jnp.float32) + bh_ref[...]


def kernel(img, w0, b0, w1, b1, w2, b2, wl1, bl1, wh, bh):
    B0 = img.shape[0]
    pad_b = (-B0) % MTB
    x = img.astype(jnp.float32)
    if pad_b:
        x = jnp.concatenate(
            [x, jnp.zeros((pad_b,) + x.shape[1:], jnp.float32)], axis=0)
    B = x.shape[0]

    x = jnp.transpose(x, (0, 2, 3, 1))
    x = jnp.pad(x, ((0, 0), (0, G - IMG_HW), (0, G - IMG_HW), (0, 0)))
    x2d = x.reshape(B * RPI, 1)

    w1b = w1.astype(jnp.bfloat16)
    w2b = w2.astype(jnp.bfloat16)
    wl1b = wl1.astype(jnp.bfloat16)
    whb = wh.astype(jnp.bfloat16)

    R = CTB * RPI
    feat_2d = pl.pallas_call(
        _conv_kernel,
        out_shape=jax.ShapeDtypeStruct((B * RPI, 16), jnp.bfloat16),
        grid=(B // CTB,),
        in_specs=[
            pl.BlockSpec((R, 1), lambda i: (i, 0)),
            pl.BlockSpec((9, 32), lambda i: (0, 0)),
            pl.BlockSpec((1, 32), lambda i: (0, 0)),
            pl.BlockSpec((9, 32, 32), lambda i: (0, 0, 0)),
            pl.BlockSpec((1, 32), lambda i: (0, 0)),
            pl.BlockSpec((9, 32, 16), lambda i: (0, 0, 0)),
            pl.BlockSpec((1, 16), lambda i: (0, 0)),
        ],
        out_specs=pl.BlockSpec((R, 16), lambda i: (i, 0)),
        scratch_shapes=[pltpu.VMEM((R - SH, 32), jnp.float32),
                        pltpu.VMEM((R - SH, 32), jnp.bfloat16),
                        pltpu.VMEM((R - 2 * SH, 32), jnp.float32),
                        pltpu.VMEM((R - 2 * SH, 32), jnp.bfloat16)],
        compiler_params=pltpu.CompilerParams(
            dimension_semantics=("parallel",)),
    )(x2d, w0, b0, w1b, b1, w2b, b2)

    feat = feat_2d.reshape(B, G, G, 16)[:, :24, :24, :]
    feat = feat.reshape(B, 24 * 24 * 16)

    heads = pl.pallas_call(
        _mlp_kernel,
        out_shape=jax.ShapeDtypeStruct((B, 2 * A_DIM), jnp.float32),
        grid=(B // MTB,),
        in_specs=[
            pl.BlockSpec((MTB, 24 * 24 * 16), lambda i: (i, 0)),
            pl.BlockSpec((24 * 24 * 16, 128), lambda i: (0, 0)),
            pl.BlockSpec((1, 128), lambda i: (0, 0)),
            pl.BlockSpec((128, 2 * A_DIM), lambda i: (0, 0)),
            pl.BlockSpec((1, 2 * A_DIM), lambda i: (0, 0)),
        ],
        out_specs=pl.BlockSpec((MTB, 2 * A_DIM), lambda i: (i, 0)),
        compiler_params=pltpu.CompilerParams(
            dimension_semantics=("parallel",),
            vmem_limit_bytes=64 * 1024 * 1024),
    )(feat, wl1b, bl1, whb, bh)

    mean = heads[:B0, :A_DIM]
    log_std = heads[:B0, A_DIM:]
    return mean, log_std


# speedup vs baseline: 7.1049x; 6.7589x over previous
import jax
import jax.numpy as jnp
from jax.experimental import pallas as pl
from jax.experimental.pallas import tpu as pltpu

A_DIM = 5
IMG_HW = 30
G = 32
XS = 32
CTB = 64
MTB = 256
KL1 = G * 24 * 16


def _band(w3, x_in, c_in, x_out, c_out):
    acc = jnp.zeros((x_in, c_in, x_out, c_out), jnp.float32)
    for kx in range(3):
        eye = jnp.eye(x_in, x_out, -kx, dtype=jnp.float32)
        acc = acc + eye[:, None, :, None] * w3[kx][None, :, None, :]
    return acc.reshape(x_in * c_in, x_out * c_out)


def _conv_kernel(x_ref, wb0_ref, b0_ref, wb1_ref, b1_ref, wb2_ref, b2_ref,
                 out_ref, xs_ref, a1_ref, a2_ref):
    M = out_ref.shape[0]

    xs_ref[pl.ds(0, M), :] = x_ref[...]
    xs_ref[pl.ds(M, 8), :] = jnp.zeros((8, xs_ref.shape[1]), jnp.bfloat16)

    acc = jnp.dot(xs_ref[pl.ds(0, M), :], wb0_ref[0],
                  preferred_element_type=jnp.float32)
    for ky in (1, 2):
        acc += jnp.dot(xs_ref[pl.ds(ky, M), :], wb0_ref[ky],
                       preferred_element_type=jnp.float32)
    a1_ref[pl.ds(0, M), :] = jnp.maximum(acc + b0_ref[...],
                                         0.0).astype(jnp.bfloat16)
    a1_ref[pl.ds(M, 8), :] = jnp.zeros((8, a1_ref.shape[1]), jnp.bfloat16)

    acc = jnp.dot(a1_ref[pl.ds(0, M), :], wb1_ref[0],
                  preferred_element_type=jnp.float32)
    for ky in (1, 2):
        acc += jnp.dot(a1_ref[pl.ds(ky, M), :], wb1_ref[ky],
                       preferred_element_type=jnp.float32)
    a2_ref[pl.ds(0, M), :] = jnp.maximum(acc + b1_ref[...],
                                         0.0).astype(jnp.bfloat16)
    a2_ref[pl.ds(M, 8), :] = jnp.zeros((8, a2_ref.shape[1]), jnp.bfloat16)

    acc = jnp.dot(a2_ref[pl.ds(0, M), :], wb2_ref[0],
                  preferred_element_type=jnp.float32)
    for ky in (1, 2):
        acc += jnp.dot(a2_ref[pl.ds(ky, M), :], wb2_ref[ky],
                       preferred_element_type=jnp.float32)
    out_ref[...] = jnp.maximum(acc + b2_ref[...], 0.0).astype(jnp.bfloat16)


def _mlp_kernel(x_ref, wl1_ref, bl1_ref, wh_ref, bh_ref, out_ref):
    h = jnp.dot(x_ref[...], wl1_ref[...], preferred_element_type=jnp.float32)
    h = jnp.maximum(h + bl1_ref[...], 0.0).astype(jnp.bfloat16)
    out_ref[...] = jnp.dot(h, wh_ref[...],
                           preferred_element_type=jnp.float32) + bh_ref[...]


def kernel(img, w0, b0, w1, b1, w2, b2, wl1, bl1, wh, bh):
    B0 = img.shape[0]
    pad_b = (-B0) % MTB
    x = img.astype(jnp.float32)
    if pad_b:
        x = jnp.concatenate(
            [x, jnp.zeros((pad_b,) + x.shape[1:], jnp.float32)], axis=0)
    B = x.shape[0]

    x = x.reshape(B, IMG_HW, IMG_HW)
    x = jnp.pad(x, ((0, 0), (0, G - IMG_HW), (0, XS - IMG_HW)))
    x2d = x.reshape(B * G, XS).astype(jnp.bfloat16)

    wb0 = jnp.stack([_band(w0[3 * ky:3 * ky + 3].reshape(3, 1, 32),
                           XS, 1, XS, 32) for ky in range(3)])
    wb1 = jnp.stack([_band(w1[3 * ky:3 * ky + 3], XS, 32, XS, 32)
                     for ky in range(3)])
    wb2 = jnp.stack([_band(w2[3 * ky:3 * ky + 3], XS, 32, 24, 16)
                     for ky in range(3)])
    wb0 = wb0.astype(jnp.bfloat16)
    wb1 = wb1.astype(jnp.bfloat16)
    wb2 = wb2.astype(jnp.bfloat16)
    b0t = jnp.tile(b0, (1, XS))
    b1t = jnp.tile(b1, (1, XS))
    b2t = jnp.tile(b2, (1, 24))

    wl1p = jnp.pad(wl1.reshape(24, 24 * 16, 128),
                   ((0, G - 24), (0, 0), (0, 0))).reshape(KL1, 128)
    wl1p = wl1p.astype(jnp.bfloat16)
    whb = wh.astype(jnp.bfloat16)

    M = CTB * G
    feat = pl.pallas_call(
        _conv_kernel,
        out_shape=jax.ShapeDtypeStruct((B * G, 24 * 16), jnp.bfloat16),
        grid=(B // CTB,),
        in_specs=[
            pl.BlockSpec((M, XS), lambda i: (i, 0)),
            pl.BlockSpec((3, XS, XS * 32), lambda i: (0, 0, 0)),
            pl.BlockSpec((1, XS * 32), lambda i: (0, 0)),
            pl.BlockSpec((3, XS * 32, XS * 32), lambda i: (0, 0, 0)),
            pl.BlockSpec((1, XS * 32), lambda i: (0, 0)),
            pl.BlockSpec((3, XS * 32, 24 * 16), lambda i: (0, 0, 0)),
            pl.BlockSpec((1, 24 * 16), lambda i: (0, 0)),
        ],
        out_specs=pl.BlockSpec((M, 24 * 16), lambda i: (i, 0)),
        scratch_shapes=[pltpu.VMEM((M + 8, XS), jnp.bfloat16),
                        pltpu.VMEM((M + 8, XS * 32), jnp.bfloat16),
                        pltpu.VMEM((M + 8, XS * 32), jnp.bfloat16)],
        compiler_params=pltpu.CompilerParams(
            dimension_semantics=("parallel",)),
    )(x2d, wb0, b0t, wb1, b1t, wb2, b2t)

    feat = feat.reshape(B, KL1)

    heads = pl.pallas_call(
        _mlp_kernel,
        out_shape=jax.ShapeDtypeStruct((B, 2 * A_DIM), jnp.float32),
        grid=(B // MTB,),
        in_specs=[
            pl.BlockSpec((MTB, KL1), lambda i: (i, 0)),
            pl.BlockSpec((KL1, 128), lambda i: (0, 0)),
            pl.BlockSpec((1, 128), lambda i: (0, 0)),
            pl.BlockSpec((128, 2 * A_DIM), lambda i: (0, 0)),
            pl.BlockSpec((1, 2 * A_DIM), lambda i: (0, 0)),
        ],
        out_specs=pl.BlockSpec((MTB, 2 * A_DIM), lambda i: (i, 0)),
        compiler_params=pltpu.CompilerParams(
            dimension_semantics=("parallel",)),
    )(feat, wl1p, bl1, whb, bh)

    mean = heads[:B0, :A_DIM]
    log_std = heads[:B0, A_DIM:]
    return mean, log_std


# speedup vs baseline: 14.1347x; 1.9894x over previous
import jax
import jax.numpy as jnp
from jax.experimental import pallas as pl
from jax.experimental.pallas import tpu as pltpu

A_DIM = 5
IMG_HW = 30
XS = 32
CTB = 64


def _band(w3, x_in, c_in, x_out, c_out):
    acc = jnp.zeros((x_in, c_in, x_out, c_out), jnp.float32)
    for kx in range(3):
        eye = jnp.eye(x_in, x_out, -kx, dtype=jnp.float32)
        acc = acc + eye[:, None, :, None] * w3[kx][None, :, None, :]
    return acc.reshape(x_in * c_in, x_out * c_out)


def _fused_kernel(x_ref, wb0_ref, b0_ref, wb1_ref, b1_ref, wb2_ref, b2_ref,
                  wl1_ref, bl1_ref, wh_ref, bh_ref,
                  out_ref, xs_ref, a1_ref, a2_ref, f_ref):
    C = out_ref.shape[0]
    M0, M1, M2 = 28 * C, 26 * C, 24 * C

    xs_ref[...] = x_ref[...].reshape(30 * C, XS)

    acc = jnp.dot(xs_ref[pl.ds(0, M0), :], wb0_ref[0],
                  preferred_element_type=jnp.float32)
    for ky in (1, 2):
        acc += jnp.dot(xs_ref[pl.ds(ky * C, M0), :], wb0_ref[ky],
                       preferred_element_type=jnp.float32)
    a1_ref[...] = jnp.maximum(acc + b0_ref[...], 0.0).astype(jnp.bfloat16)

    acc = jnp.dot(a1_ref[pl.ds(0, M1), :], wb1_ref[0],
                  preferred_element_type=jnp.float32)
    for ky in (1, 2):
        acc += jnp.dot(a1_ref[pl.ds(ky * C, M1), :], wb1_ref[ky],
                       preferred_element_type=jnp.float32)
    a2_ref[...] = jnp.maximum(acc + b1_ref[...], 0.0).astype(jnp.bfloat16)

    acc = jnp.dot(a2_ref[pl.ds(0, M2), :], wb2_ref[0],
                  preferred_element_type=jnp.float32)
    for ky in (1, 2):
        acc += jnp.dot(a2_ref[pl.ds(ky * C, M2), :], wb2_ref[ky],
                       preferred_element_type=jnp.float32)
    f_ref[...] = jnp.maximum(acc + b2_ref[...], 0.0).astype(jnp.bfloat16)

    h = jnp.dot(f_ref[pl.ds(0, C), :], wl1_ref[0],
                preferred_element_type=jnp.float32)
    for y in range(1, 24):
        h += jnp.dot(f_ref[pl.ds(y * C, C), :], wl1_ref[y],
                     preferred_element_type=jnp.float32)
    h = jnp.maximum(h + bl1_ref[...], 0.0).astype(jnp.bfloat16)

    out_ref[...] = jnp.dot(h, wh_ref[...],
                           preferred_element_type=jnp.float32) + bh_ref[...]


def kernel(img, w0, b0, w1, b1, w2, b2, wl1, bl1, wh, bh):
    B0 = img.shape[0]
    pad_b = (-B0) % CTB
    x = img.astype(jnp.float32)
    if pad_b:
        x = jnp.concatenate(
            [x, jnp.zeros((pad_b,) + x.shape[1:], jnp.float32)], axis=0)
    B = x.shape[0]

    x = jnp.transpose(x.reshape(B, IMG_HW, IMG_HW), (1, 0, 2))
    x = jnp.pad(x, ((0, 0), (0, 0), (0, XS - IMG_HW)))
    x3 = x.astype(jnp.bfloat16)

    wb0 = jnp.stack([_band(w0[3 * ky:3 * ky + 3].reshape(3, 1, 32),
                           XS, 1, XS, 32) for ky in range(3)])
    wb1 = jnp.stack([_band(w1[3 * ky:3 * ky + 3], XS, 32, XS, 32)
                     for ky in range(3)])
    wb2 = jnp.stack([_band(w2[3 * ky:3 * ky + 3], XS, 32, 24, 16)
                     for ky in range(3)])
    wb0 = wb0.astype(jnp.bfloat16)
    wb1 = wb1.astype(jnp.bfloat16)
    wb2 = wb2.astype(jnp.bfloat16)
    b0t = jnp.tile(b0, (1, XS))
    b1t = jnp.tile(b1, (1, XS))
    b2t = jnp.tile(b2, (1, 24))

    wl1r = wl1.reshape(24, 24 * 16, 128).astype(jnp.bfloat16)
    whb = wh.astype(jnp.bfloat16)

    heads = pl.pallas_call(
        _fused_kernel,
        out_shape=jax.ShapeDtypeStruct((B, 2 * A_DIM), jnp.float32),
        grid=(B // CTB,),
        in_specs=[
            pl.BlockSpec((30, CTB, XS), lambda i: (0, i, 0)),
            pl.BlockSpec((3, XS, XS * 32), lambda i: (0, 0, 0)),
            pl.BlockSpec((1, XS * 32), lambda i: (0, 0)),
            pl.BlockSpec((3, XS * 32, XS * 32), lambda i: (0, 0, 0)),
            pl.BlockSpec((1, XS * 32), lambda i: (0, 0)),
            pl.BlockSpec((3, XS * 32, 24 * 16), lambda i: (0, 0, 0)),
            pl.BlockSpec((1, 24 * 16), lambda i: (0, 0)),
            pl.BlockSpec((24, 24 * 16, 128), lambda i: (0, 0, 0)),
            pl.BlockSpec((1, 128), lambda i: (0, 0)),
            pl.BlockSpec((128, 2 * A_DIM), lambda i: (0, 0)),
            pl.BlockSpec((1, 2 * A_DIM), lambda i: (0, 0)),
        ],
        out_specs=pl.BlockSpec((CTB, 2 * A_DIM), lambda i: (i, 0)),
        scratch_shapes=[pltpu.VMEM((30 * CTB, XS), jnp.bfloat16),
                        pltpu.VMEM((28 * CTB, XS * 32), jnp.bfloat16),
                        pltpu.VMEM((26 * CTB, XS * 32), jnp.bfloat16),
                        pltpu.VMEM((24 * CTB, 24 * 16), jnp.bfloat16)],
        compiler_params=pltpu.CompilerParams(
            dimension_semantics=("parallel",)),
    )(x3, wb0, b0t, wb1, b1t, wb2, b2t, wl1r, bl1, whb, bh)

    mean = heads[:B0, :A_DIM]
    log_std = heads[:B0, A_DIM:]
    return mean, log_std


# speedup vs baseline: 18.7748x; 1.3283x over previous
import jax
import jax.numpy as jnp
from jax.experimental import pallas as pl
from jax.experimental.pallas import tpu as pltpu

A_DIM = 5
IMG_HW = 30
XS = 32
CTB = 64
KS1 = (0, 256, 512, 640)


def _band(w3, x_in, c_in, x_out, c_out):
    acc = jnp.zeros((x_in, c_in, x_out, c_out), jnp.float32)
    for kx in range(3):
        eye = jnp.eye(x_in, x_out, -kx, dtype=jnp.float32)
        acc = acc + eye[:, None, :, None] * w3[kx][None, :, None, :]
    return acc.reshape(x_in * c_in, x_out * c_out)


def _fused_kernel(x_ref, wb0_ref, b0_ref, wb1_ref, b1_ref, wb2_ref, b2_ref,
                  wl1_ref, bl1_ref, wh_ref, bh_ref,
                  out_ref, xs_ref, a1_ref, a2_ref, f_ref):
    C = out_ref.shape[0]
    M0, M1, M2 = 28 * C, 26 * C, 24 * C

    xs_ref[...] = x_ref[...].reshape(30 * C, XS)

    acc = jnp.dot(xs_ref[pl.ds(0, M0), :], wb0_ref[0],
                  preferred_element_type=jnp.float32)
    for ky in (1, 2):
        acc += jnp.dot(xs_ref[pl.ds(ky * C, M0), :], wb0_ref[ky],
                       preferred_element_type=jnp.float32)
    a1_ref[...] = jnp.maximum(acc + b0_ref[...], 0.0).astype(jnp.bfloat16)

    for j in range(4):
        s = KS1[j]
        acc = jnp.dot(a1_ref[pl.ds(0, M1), pl.ds(s, 384)], wb1_ref[0, j],
                      preferred_element_type=jnp.float32)
        for ky in (1, 2):
            acc += jnp.dot(a1_ref[pl.ds(ky * C, M1), pl.ds(s, 384)],
                           wb1_ref[ky, j],
                           preferred_element_type=jnp.float32)
        a2_ref[:, pl.ds(256 * j, 256)] = jnp.maximum(
            acc + b1_ref[:, pl.ds(256 * j, 256)], 0.0).astype(jnp.bfloat16)

    acc = jnp.dot(a2_ref[pl.ds(0, M2), :], wb2_ref[0],
                  preferred_element_type=jnp.float32)
    for ky in (1, 2):
        acc += jnp.dot(a2_ref[pl.ds(ky * C, M2), :], wb2_ref[ky],
                       preferred_element_type=jnp.float32)
    f_ref[...] = jnp.maximum(acc + b2_ref[...], 0.0).astype(jnp.bfloat16)

    h = jnp.dot(f_ref[pl.ds(0, C), :], wl1_ref[0],
                preferred_element_type=jnp.float32)
    for y in range(1, 24):
        h += jnp.dot(f_ref[pl.ds(y * C, C), :], wl1_ref[y],
                     preferred_element_type=jnp.float32)
    h = jnp.maximum(h + bl1_ref[...], 0.0).astype(jnp.bfloat16)

    out_ref[...] = jnp.dot(h, wh_ref[...],
                           preferred_element_type=jnp.float32) + bh_ref[...]


def kernel(img, w0, b0, w1, b1, w2, b2, wl1, bl1, wh, bh):
    B0 = img.shape[0]
    pad_b = (-B0) % CTB
    x = img.astype(jnp.float32)
    if pad_b:
        x = jnp.concatenate(
            [x, jnp.zeros((pad_b,) + x.shape[1:], jnp.float32)], axis=0)
    B = x.shape[0]

    x = jnp.transpose(x.reshape(B, IMG_HW, IMG_HW), (1, 0, 2))
    x = jnp.pad(x, ((0, 0), (0, 0), (0, XS - IMG_HW)))
    x3 = x.astype(jnp.bfloat16)

    wb0 = jnp.stack([_band(w0[3 * ky:3 * ky + 3].reshape(3, 1, 32),
                           XS, 1, XS, 32) for ky in range(3)])
    wb1f = [_band(w1[3 * ky:3 * ky + 3], XS, 32, XS, 32) for ky in range(3)]
    wb1 = jnp.stack([jnp.stack([wb1f[ky][KS1[j]:KS1[j] + 384,
                                         256 * j:256 * j + 256]
                                for j in range(4)]) for ky in range(3)])
    wb2 = jnp.stack([_band(w2[3 * ky:3 * ky + 3], XS, 32, 24, 16)
                     for ky in range(3)])
    wb0 = wb0.astype(jnp.bfloat16)
    wb1 = wb1.astype(jnp.bfloat16)
    wb2 = wb2.astype(jnp.bfloat16)
    b0t = jnp.tile(b0, (1, XS))
    b1t = jnp.tile(b1, (1, XS))
    b2t = jnp.tile(b2, (1, 24))

    wl1r = wl1.reshape(24, 24 * 16, 128).astype(jnp.bfloat16)
    whb = wh.astype(jnp.bfloat16)

    heads = pl.pallas_call(
        _fused_kernel,
        out_shape=jax.ShapeDtypeStruct((B, 2 * A_DIM), jnp.float32),
        grid=(B // CTB,),
        in_specs=[
            pl.BlockSpec((30, CTB, XS), lambda i: (0, i, 0)),
            pl.BlockSpec((3, XS, XS * 32), lambda i: (0, 0, 0)),
            pl.BlockSpec((1, XS * 32), lambda i: (0, 0)),
            pl.BlockSpec((3, 4, 384, 256), lambda i: (0, 0, 0, 0)),
            pl.BlockSpec((1, XS * 32), lambda i: (0, 0)),
            pl.BlockSpec((3, XS * 32, 24 * 16), lambda i: (0, 0, 0)),
            pl.BlockSpec((1, 24 * 16), lambda i: (0, 0)),
            pl.BlockSpec((24, 24 * 16, 128), lambda i: (0, 0, 0)),
            pl.BlockSpec((1, 128), lambda i: (0, 0)),
            pl.BlockSpec((128, 2 * A_DIM), lambda i: (0, 0)),
            pl.BlockSpec((1, 2 * A_DIM), lambda i: (0, 0)),
        ],
        out_specs=pl.BlockSpec((CTB, 2 * A_DIM), lambda i: (i, 0)),
        scratch_shapes=[pltpu.VMEM((30 * CTB, XS), jnp.bfloat16),
                        pltpu.VMEM((28 * CTB, XS * 32), jnp.bfloat16),
                        pltpu.VMEM((26 * CTB, XS * 32), jnp.bfloat16),
                        pltpu.VMEM((24 * CTB, 24 * 16), jnp.bfloat16)],
        compiler_params=pltpu.CompilerParams(
            dimension_semantics=("parallel",)),
    )(x3, wb0, b0t, wb1, b1t, wb2, b2t, wl1r, bl1, whb, bh)

    mean = heads[:B0, :A_DIM]
    log_std = heads[:B0, A_DIM:]
    return mean, log_std


# speedup vs baseline: 18.8239x; 1.0026x over previous
import jax
import jax.numpy as jnp
from jax.experimental import pallas as pl
from jax.experimental.pallas import tpu as pltpu

A_DIM = 5
IMG_HW = 30
XS = 32
CTB = 128
KS1 = (0, 256, 512, 640)


def _band(w3, x_in, c_in, x_out, c_out):
    acc = jnp.zeros((x_in, c_in, x_out, c_out), jnp.float32)
    for kx in range(3):
        eye = jnp.eye(x_in, x_out, -kx, dtype=jnp.float32)
        acc = acc + eye[:, None, :, None] * w3[kx][None, :, None, :]
    return acc.reshape(x_in * c_in, x_out * c_out)


def _fused_kernel(x_ref, wb0_ref, b0_ref, wb1_ref, b1_ref, wb2_ref, b2_ref,
                  wl1_ref, bl1_ref, wh_ref, bh_ref,
                  out_ref, xs_ref, a1_ref, a2_ref, f_ref):
    C = out_ref.shape[0]
    M0, M1, M2 = 28 * C, 26 * C, 24 * C

    xs_ref[...] = x_ref[...].reshape(30 * C, XS)

    acc = jnp.dot(xs_ref[pl.ds(0, M0), :], wb0_ref[0],
                  preferred_element_type=jnp.float32)
    for ky in (1, 2):
        acc += jnp.dot(xs_ref[pl.ds(ky * C, M0), :], wb0_ref[ky],
                       preferred_element_type=jnp.float32)
    a1_ref[...] = jnp.maximum(acc + b0_ref[...], 0.0).astype(jnp.bfloat16)

    for j in range(4):
        s = KS1[j]
        acc = jnp.dot(a1_ref[pl.ds(0, M1), pl.ds(s, 384)], wb1_ref[0, j],
                      preferred_element_type=jnp.float32)
        for ky in (1, 2):
            acc += jnp.dot(a1_ref[pl.ds(ky * C, M1), pl.ds(s, 384)],
                           wb1_ref[ky, j],
                           preferred_element_type=jnp.float32)
        a2_ref[:, pl.ds(256 * j, 256)] = jnp.maximum(
            acc + b1_ref[:, pl.ds(256 * j, 256)], 0.0).astype(jnp.bfloat16)

    acc = jnp.dot(a2_ref[pl.ds(0, M2), :], wb2_ref[0],
                  preferred_element_type=jnp.float32)
    for ky in (1, 2):
        acc += jnp.dot(a2_ref[pl.ds(ky * C, M2), :], wb2_ref[ky],
                       preferred_element_type=jnp.float32)
    f_ref[...] = jnp.maximum(acc + b2_ref[...], 0.0).astype(jnp.bfloat16)

    h = jnp.dot(f_ref[pl.ds(0, C), :], wl1_ref[0],
                preferred_element_type=jnp.float32)
    for y in range(1, 24):
        h += jnp.dot(f_ref[pl.ds(y * C, C), :], wl1_ref[y],
                     preferred_element_type=jnp.float32)
    h = jnp.maximum(h + bl1_ref[...], 0.0).astype(jnp.bfloat16)

    out_ref[...] = jnp.dot(h, wh_ref[...],
                           preferred_element_type=jnp.float32) + bh_ref[...]


def kernel(img, w0, b0, w1, b1, w2, b2, wl1, bl1, wh, bh):
    B0 = img.shape[0]
    pad_b = (-B0) % CTB
    x = img.astype(jnp.float32)
    if pad_b:
        x = jnp.concatenate(
            [x, jnp.zeros((pad_b,) + x.shape[1:], jnp.float32)], axis=0)
    B = x.shape[0]

    x = jnp.transpose(x.reshape(B, IMG_HW, IMG_HW), (1, 0, 2))
    x = jnp.pad(x, ((0, 0), (0, 0), (0, XS - IMG_HW)))
    x3 = x.astype(jnp.bfloat16)

    wb0 = jnp.stack([_band(w0[3 * ky:3 * ky + 3].reshape(3, 1, 32),
                           XS, 1, XS, 32) for ky in range(3)])
    wb1f = [_band(w1[3 * ky:3 * ky + 3], XS, 32, XS, 32) for ky in range(3)]
    wb1 = jnp.stack([jnp.stack([wb1f[ky][KS1[j]:KS1[j] + 384,
                                         256 * j:256 * j + 256]
                                for j in range(4)]) for ky in range(3)])
    wb2 = jnp.stack([_band(w2[3 * ky:3 * ky + 3], XS, 32, 24, 16)
                     for ky in range(3)])
    wb0 = wb0.astype(jnp.bfloat16)
    wb1 = wb1.astype(jnp.bfloat16)
    wb2 = wb2.astype(jnp.bfloat16)
    b0t = jnp.tile(b0, (1, XS))
    b1t = jnp.tile(b1, (1, XS))
    b2t = jnp.tile(b2, (1, 24))

    wl1r = wl1.reshape(24, 24 * 16, 128).astype(jnp.bfloat16)
    whb = wh.astype(jnp.bfloat16)

    heads = pl.pallas_call(
        _fused_kernel,
        out_shape=jax.ShapeDtypeStruct((B, 2 * A_DIM), jnp.float32),
        grid=(B // CTB,),
        in_specs=[
            pl.BlockSpec((30, CTB, XS), lambda i: (0, i, 0)),
            pl.BlockSpec((3, XS, XS * 32), lambda i: (0, 0, 0)),
            pl.BlockSpec((1, XS * 32), lambda i: (0, 0)),
            pl.BlockSpec((3, 4, 384, 256), lambda i: (0, 0, 0, 0)),
            pl.BlockSpec((1, XS * 32), lambda i: (0, 0)),
            pl.BlockSpec((3, XS * 32, 24 * 16), lambda i: (0, 0, 0)),
            pl.BlockSpec((1, 24 * 16), lambda i: (0, 0)),
            pl.BlockSpec((24, 24 * 16, 128), lambda i: (0, 0, 0)),
            pl.BlockSpec((1, 128), lambda i: (0, 0)),
            pl.BlockSpec((128, 2 * A_DIM), lambda i: (0, 0)),
            pl.BlockSpec((1, 2 * A_DIM), lambda i: (0, 0)),
        ],
        out_specs=pl.BlockSpec((CTB, 2 * A_DIM), lambda i: (i, 0)),
        scratch_shapes=[pltpu.VMEM((30 * CTB, XS), jnp.bfloat16),
                        pltpu.VMEM((28 * CTB, XS * 32), jnp.bfloat16),
                        pltpu.VMEM((26 * CTB, XS * 32), jnp.bfloat16),
                        pltpu.VMEM((24 * CTB, 24 * 16), jnp.bfloat16)],
        compiler_params=pltpu.CompilerParams(
            dimension_semantics=("parallel",)),
    )(x3, wb0, b0t, wb1, b1t, wb2, b2t, wl1r, bl1, whb, bh)

    mean = heads[:B0, :A_DIM]
    log_std = heads[:B0, A_DIM:]
    return mean, log_std


# speedup vs baseline: 23.6341x; 1.2555x over previous
import jax
import jax.numpy as jnp
from jax.experimental import pallas as pl
from jax.experimental.pallas import tpu as pltpu

A_DIM = 5
IMG_HW = 30
XS = 32
CTB = 128
KS1 = (0, 256, 512, 640)


def _band(w3, x_in, c_in, x_out, c_out):
    acc = jnp.zeros((x_in, c_in, x_out, c_out), jnp.float32)
    for kx in range(3):
        eye = jnp.eye(x_in, x_out, -kx, dtype=jnp.float32)
        acc = acc + eye[:, None, :, None] * w3[kx][None, :, None, :]
    return acc.reshape(x_in * c_in, x_out * c_out)


def _fused_kernel(x_ref, wb0_ref, b0_ref, wb1_ref, b1_ref, wb2_ref, b2_ref,
                  wl1_ref, bl1_ref, wh_ref, bh_ref,
                  out_ref, a1_ref, a2_ref, f_ref):
    C = out_ref.shape[0]
    M0, M1, M2 = 28 * C, 26 * C, 24 * C

    xv = x_ref[...].reshape(M0, 96)
    acc = jnp.dot(xv, wb0_ref[...], preferred_element_type=jnp.float32)
    a1_ref[...] = jnp.maximum(acc + b0_ref[...], 0.0).astype(jnp.bfloat16)

    for j in range(4):
        s = KS1[j]
        acc = jnp.dot(a1_ref[pl.ds(0, M1), pl.ds(s, 384)], wb1_ref[0, j],
                      preferred_element_type=jnp.float32)
        for ky in (1, 2):
            acc += jnp.dot(a1_ref[pl.ds(ky * C, M1), pl.ds(s, 384)],
                           wb1_ref[ky, j],
                           preferred_element_type=jnp.float32)
        a2_ref[:, pl.ds(256 * j, 256)] = jnp.maximum(
            acc + b1_ref[:, pl.ds(256 * j, 256)], 0.0).astype(jnp.bfloat16)

    for g in range(3):
        s = 256 * g
        acc = jnp.dot(a2_ref[pl.ds(0, M2), pl.ds(s, 384)], wb2_ref[0, g],
                      preferred_element_type=jnp.float32)
        for ky in (1, 2):
            acc += jnp.dot(a2_ref[pl.ds(ky * C, M2), pl.ds(s, 384)],
                           wb2_ref[ky, g],
                           preferred_element_type=jnp.float32)
        f_ref[:, pl.ds(128 * g, 128)] = jnp.maximum(
            acc + b2_ref[:, pl.ds(128 * g, 128)], 0.0).astype(jnp.bfloat16)

    h = jnp.dot(f_ref[pl.ds(0, C), :], wl1_ref[0],
                preferred_element_type=jnp.float32)
    for y in range(1, 24):
        h += jnp.dot(f_ref[pl.ds(y * C, C), :], wl1_ref[y],
                     preferred_element_type=jnp.float32)
    h = jnp.maximum(h + bl1_ref[...], 0.0).astype(jnp.bfloat16)

    out_ref[...] = jnp.dot(h, wh_ref[...],
                           preferred_element_type=jnp.float32) + bh_ref[...]


def kernel(img, w0, b0, w1, b1, w2, b2, wl1, bl1, wh, bh):
    B0 = img.shape[0]
    pad_b = (-B0) % CTB
    x = img.astype(jnp.float32)
    if pad_b:
        x = jnp.concatenate(
            [x, jnp.zeros((pad_b,) + x.shape[1:], jnp.float32)], axis=0)
    B = x.shape[0]

    x = jnp.transpose(x.reshape(B, IMG_HW, IMG_HW), (1, 0, 2))
    x = jnp.pad(x, ((0, 0), (0, 0), (0, XS - IMG_HW)))
    x3 = x.astype(jnp.bfloat16)
    x3c = jnp.concatenate([x3[0:28], x3[1:29], x3[2:30]], axis=2)

    wb0 = jnp.concatenate([_band(w0[3 * ky:3 * ky + 3].reshape(3, 1, 32),
                                 XS, 1, XS, 32) for ky in range(3)], axis=0)
    wb1f = [_band(w1[3 * ky:3 * ky + 3], XS, 32, XS, 32) for ky in range(3)]
    wb1 = jnp.stack([jnp.stack([wb1f[ky][KS1[j]:KS1[j] + 384,
                                         256 * j:256 * j + 256]
                                for j in range(4)]) for ky in range(3)])
    wb2f = [_band(w2[3 * ky:3 * ky + 3], XS, 32, 24, 16) for ky in range(3)]
    wb2 = jnp.stack([jnp.stack([wb2f[ky][256 * g:256 * g + 384,
                                         128 * g:128 * g + 128]
                                for g in range(3)]) for ky in range(3)])
    wb0 = wb0.astype(jnp.bfloat16)
    wb1 = wb1.astype(jnp.bfloat16)
    wb2 = wb2.astype(jnp.bfloat16)
    b0t = jnp.tile(b0, (1, XS))
    b1t = jnp.tile(b1, (1, XS))
    b2t = jnp.tile(b2, (1, 24))

    wl1r = wl1.reshape(24, 24 * 16, 128).astype(jnp.bfloat16)
    whb = wh.astype(jnp.bfloat16)

    heads = pl.pallas_call(
        _fused_kernel,
        out_shape=jax.ShapeDtypeStruct((B, 2 * A_DIM), jnp.float32),
        grid=(B // CTB,),
        in_specs=[
            pl.BlockSpec((28, CTB, 96), lambda i: (0, i, 0)),
            pl.BlockSpec((96, XS * 32), lambda i: (0, 0)),
            pl.BlockSpec((1, XS * 32), lambda i: (0, 0)),
            pl.BlockSpec((3, 4, 384, 256), lambda i: (0, 0, 0, 0)),
            pl.BlockSpec((1, XS * 32), lambda i: (0, 0)),
            pl.BlockSpec((3, 3, 384, 128), lambda i: (0, 0, 0, 0)),
            pl.BlockSpec((1, 24 * 16), lambda i: (0, 0)),
            pl.BlockSpec((24, 24 * 16, 128), lambda i: (0, 0, 0)),
            pl.BlockSpec((1, 128), lambda i: (0, 0)),
            pl.BlockSpec((128, 2 * A_DIM), lambda i: (0, 0)),
            pl.BlockSpec((1, 2 * A_DIM), lambda i: (0, 0)),
        ],
        out_specs=pl.BlockSpec((CTB, 2 * A_DIM), lambda i: (i, 0)),
        scratch_shapes=[pltpu.VMEM((28 * CTB, XS * 32), jnp.bfloat16),
                        pltpu.VMEM((26 * CTB, XS * 32), jnp.bfloat16),
                        pltpu.VMEM((24 * CTB, 24 * 16), jnp.bfloat16)],
        compiler_params=pltpu.CompilerParams(
            dimension_semantics=("parallel",)),
    )(x3c, wb0, b0t, wb1, b1t, wb2, b2t, wl1r, bl1, whb, bh)

    mean = heads[:B0, :A_DIM]
    log_std = heads[:B0, A_DIM:]
    return mean, log_std
